# baseline (speedup 1.0000x reference)
import functools

import jax
import jax.numpy as jnp
from jax import lax
from jax.experimental import pallas as pl
from jax.experimental.pallas import tpu as pltpu
from jax.experimental.pallas import tpu_sc as plsc

N = 100000
NP = 102400
E = 3200000
EP = 3211264
EROWS = EP // 128
BR = 392
CPT = NP // 16
NB_TC = NP // 1024
DUMMY = 100000

_f32 = jnp.float32


def _mesh():
    return plsc.VectorSubcoreMesh(core_axis_name="c", subcore_axis_name="s")


@functools.partial(
    pl.kernel,
    mesh=_mesh(),
    out_type=[jax.ShapeDtypeStruct((NP,), _f32),
              jax.ShapeDtypeStruct((NP,), _f32)],
    scratch_types=[pltpu.VMEM((BR, 128), jnp.int32),
                   pltpu.VMEM((128,), _f32),
                   pltpu.VMEM_SHARED((NP,), _f32),
                   pltpu.SemaphoreType.DMA],
    compiler_params=pltpu.CompilerParams(use_tc_tiling_on_sc=False),
)
def _deg_kernel(src_hbm, dst_hbm, zeros_hbm, outdeg_hbm, indeg_hbm,
                idx_v, ones_v, acc, sem):
    c = lax.axis_index("c")
    s = lax.axis_index("s")
    for k in range(8):
        ones_v[pl.ds(k * 16, 16)] = jnp.ones((16,), _f32)
    pltpu.sync_copy(zeros_hbm, acc.at[pl.ds(s * CPT, CPT)])
    plsc.subcore_barrier()

    rpt = EROWS // 16
    nb = rpt // BR

    def one_pass(idx_hbm):
        for b in range(nb):
            row0 = s * rpt + b * BR
            pltpu.sync_copy(idx_hbm.at[pl.ds(row0, BR)], idx_v)

            def body(j, carry):
                pltpu.sync_copy(ones_v, acc.at[idx_v.at[j]], add=True)
                return carry

            lax.fori_loop(0, BR, body, 0)

    @pl.when(c == 0)
    def _():
        one_pass(src_hbm)

    @pl.when(c == 1)
    def _():
        one_pass(dst_hbm)

    plsc.subcore_barrier()

    @pl.when(c == 0)
    def _():
        pltpu.sync_copy(acc.at[pl.ds(s * CPT, CPT)],
                        outdeg_hbm.at[pl.ds(s * CPT, CPT)])

    @pl.when(c == 1)
    def _():
        pltpu.sync_copy(acc.at[pl.ds(s * CPT, CPT)],
                        indeg_hbm.at[pl.ds(s * CPT, CPT)])


BRS = 98


@functools.partial(
    pl.kernel,
    mesh=_mesh(),
    out_type=jax.ShapeDtypeStruct((8 * NP,), _f32),
    scratch_types=[pltpu.VMEM((4 * BRS, 128), jnp.int32),
                   pltpu.VMEM((BRS, 128), jnp.int32),
                   pltpu.VMEM((128,), _f32),
                   pltpu.VMEM_SHARED((NP,), _f32),
                   pltpu.VMEM_SHARED((NP,), _f32),
                   pltpu.VMEM_SHARED((NP,), _f32),
                   pltpu.VMEM_SHARED((NP,), _f32),
                   pltpu.SemaphoreType.DMA],
    compiler_params=pltpu.CompilerParams(use_tc_tiling_on_sc=False),
)
def _seg_kernel(h_hbm, w_hbm, dst_hbm, zeros_hbm, out_hbm,
                iw, idx_d, col_v, a0, a1, a2, a3, sem):
    c = lax.axis_index("c")
    s = lax.axis_index("s")
    accs = (a0, a1, a2, a3)
    for a in accs:
        pltpu.sync_copy(zeros_hbm, a.at[pl.ds(s * CPT, CPT)])
    plsc.subcore_barrier()

    rpc = EROWS // 2
    rpt = rpc // 16
    nb = rpt // BRS

    for b in range(nb):
        row0 = c * rpc + s * rpt + b * BRS
        for cc in range(4):
            pltpu.sync_copy(w_hbm.at[pl.ds(cc * EROWS + row0, BRS)],
                            iw.at[pl.ds(cc * BRS, BRS)])
        pltpu.sync_copy(dst_hbm.at[pl.ds(row0, BRS)], idx_d)

        def body(j, carry):
            for cc in range(4):
                pltpu.async_copy(h_hbm.at[iw.at[cc * BRS + j]], col_v,
                                 sem).wait()
                pltpu.sync_copy(col_v, accs[cc].at[idx_d.at[j]], add=True)
            return carry

        lax.fori_loop(0, BRS, body, 0)

    plsc.subcore_barrier()

    for k in range(4):
        pltpu.sync_copy(
            accs[k].at[pl.ds(s * CPT, CPT)],
            out_hbm.at[pl.ds((c * 4 + k) * NP + s * CPT, CPT)])


def _seg16(h16, widx4, dst2, zeros_cpt, n_groups):
    hflat = h16.reshape(NP * 16)
    cols = []
    for g in range(n_groups):
        flat = _seg_kernel(hflat, widx4[g], dst2, zeros_cpt)
        parts = flat.reshape(8, NP)
        for cc in range(4):
            cols.append((parts[cc] + parts[4 + cc]).reshape(NP, 1))
    return jnp.concatenate(cols, axis=1)


def _norm(d):
    return jnp.where(d > 0.0, lax.rsqrt(d), 0.0)


def _t1_body(x_ref, od_ref, o_ref):
    h = x_ref[...] * _norm(od_ref[...])
    o_ref[...] = jnp.pad(h, ((0, 0), (0, 8)))


def _t1(xp, od2):
    return pl.pallas_call(
        _t1_body,
        grid=(NB_TC,),
        in_specs=[pl.BlockSpec((1024, 8), lambda i: (i, 0)),
                  pl.BlockSpec((1024, 1), lambda i: (i, 0))],
        out_specs=pl.BlockSpec((1024, 16), lambda i: (i, 0)),
        out_shape=jax.ShapeDtypeStruct((NP, 16), _f32),
    )(xp, od2)


def _t2_body(m_ref, id_ref, od_ref, w_ref, bias_ref, o_ref):
    m = m_ref[...] * _norm(id_ref[...])
    h = jnp.dot(m, w_ref[...], preferred_element_type=_f32) + bias_ref[...]
    o_ref[...] = jnp.maximum(h, 0.0) * _norm(od_ref[...])


def _t2(m8, id2, od2, w, bias, win, wout):
    return pl.pallas_call(
        _t2_body,
        grid=(NB_TC,),
        in_specs=[pl.BlockSpec((1024, win), lambda i: (i, 0)),
                  pl.BlockSpec((1024, 1), lambda i: (i, 0)),
                  pl.BlockSpec((1024, 1), lambda i: (i, 0)),
                  pl.BlockSpec((win, wout), lambda i: (0, 0)),
                  pl.BlockSpec((1, wout), lambda i: (0, 0))],
        out_specs=pl.BlockSpec((1024, wout), lambda i: (i, 0)),
        out_shape=jax.ShapeDtypeStruct((NP, wout), _f32),
    )(m8, id2, od2, w, bias)


def _t3_body(m_ref, id_ref, od_ref, w_ref, bias_ref, lo_ref, hi_ref):
    m = m_ref[...] * _norm(id_ref[...])
    h = jnp.dot(m, w_ref[...], preferred_element_type=_f32) + bias_ref[...]
    h = jnp.maximum(h, 0.0) * _norm(od_ref[...])
    lo_ref[...] = h[:, :16]
    hi_ref[...] = h[:, 16:]


def _t3(m16, id2, od2, w2, b2):
    return pl.pallas_call(
        _t3_body,
        grid=(NB_TC,),
        in_specs=[pl.BlockSpec((1024, 16), lambda i: (i, 0)),
                  pl.BlockSpec((1024, 1), lambda i: (i, 0)),
                  pl.BlockSpec((1024, 1), lambda i: (i, 0)),
                  pl.BlockSpec((16, 32), lambda i: (0, 0)),
                  pl.BlockSpec((1, 32), lambda i: (0, 0))],
        out_specs=[pl.BlockSpec((1024, 16), lambda i: (i, 0)),
                   pl.BlockSpec((1024, 16), lambda i: (i, 0))],
        out_shape=[jax.ShapeDtypeStruct((NP, 16), _f32),
                   jax.ShapeDtypeStruct((NP, 16), _f32)],
    )(m16, id2, od2, w2, b2)


def _t4_body(m_ref, id_ref, w3_ref, b3_ref,
             f1w_ref, f1b_ref, f2w_ref, f2b_ref, f3w_ref, f3b_ref,
             f4w_ref, f4b_ref, o_ref):
    i = pl.program_id(0)
    m = m_ref[...] * _norm(id_ref[...])
    h3 = jnp.dot(m, w3_ref[...], preferred_element_type=_f32) + b3_ref[...]
    p = jnp.maximum(h3[:, :64], h3[:, 64:])
    h = jnp.maximum(jnp.dot(p, f1w_ref[...], preferred_element_type=_f32)
                    + f1b_ref[...], 0.0)
    h = jnp.maximum(jnp.dot(h, f2w_ref[...], preferred_element_type=_f32)
                    + f2b_ref[...], 0.0)
    h = jnp.maximum(jnp.dot(h, f3w_ref[...], preferred_element_type=_f32)
                    + f3b_ref[...], 0.0)
    h4 = jnp.dot(h, f4w_ref[...], preferred_element_type=_f32) + f4b_ref[...]
    rid = i * 1024 + lax.broadcasted_iota(jnp.int32, (1024, 16), 0)
    h4 = jnp.where(rid < N, h4, 0.0)
    part = jnp.sum(h4, axis=0, keepdims=True)

    @pl.when(i == 0)
    def _():
        o_ref[...] = jnp.zeros_like(o_ref)

    o_ref[...] += part

    @pl.when(i == NB_TC - 1)
    def _():
        o_ref[...] = o_ref[...] * (1.0 / N)


def _t4(m32, id2, w3p, b3p, f1w, f1b, f2w, f2b, f3w, f3b, f4wp, f4bp):
    return pl.pallas_call(
        _t4_body,
        grid=(NB_TC,),
        in_specs=[pl.BlockSpec((1024, 32), lambda i: (i, 0)),
                  pl.BlockSpec((1024, 1), lambda i: (i, 0)),
                  pl.BlockSpec((32, 128), lambda i: (0, 0)),
                  pl.BlockSpec((1, 128), lambda i: (0, 0)),
                  pl.BlockSpec((64, 128), lambda i: (0, 0)),
                  pl.BlockSpec((1, 128), lambda i: (0, 0)),
                  pl.BlockSpec((128, 64), lambda i: (0, 0)),
                  pl.BlockSpec((1, 64), lambda i: (0, 0)),
                  pl.BlockSpec((64, 32), lambda i: (0, 0)),
                  pl.BlockSpec((1, 32), lambda i: (0, 0)),
                  pl.BlockSpec((32, 16), lambda i: (0, 0)),
                  pl.BlockSpec((1, 16), lambda i: (0, 0))],
        out_specs=pl.BlockSpec((1, 16), lambda i: (0, 0)),
        out_shape=jax.ShapeDtypeStruct((1, 16), _f32),
    )(m32, id2, w3p, b3p, f1w, f1b, f2w, f2b, f3w, f3b, f4wp, f4bp)


def kernel(edge_index, n_feat, W1, b1, W2, b2, W3, b3,
           fc1W, fc1b, fc2W, fc2b, fc3W, fc3b, fc4W, fc4b):
    src = edge_index[0]
    dst = edge_index[1]
    pad = jnp.full((EP - E,), DUMMY, jnp.int32)
    src1 = jnp.concatenate([src, pad])
    dst1 = jnp.concatenate([dst, pad])
    src2 = src1.reshape(EROWS, 128)
    dst2 = dst1.reshape(EROWS, 128)
    src16 = src2 * 16
    widx4 = [jnp.concatenate([src16 + (4 * g + cc) for cc in range(4)],
                             axis=0) for g in range(4)]
    zeros_cpt = jnp.zeros((CPT,), _f32)
    xp = jnp.zeros((NP, 8), _f32).at[:N].set(n_feat)

    od, idg = _deg_kernel(src2, dst2, zeros_cpt)
    od2 = od.reshape(NP, 1)
    id2 = idg.reshape(NP, 1)

    h0 = _t1(xp, od2)
    m8 = _seg16(h0, widx4, dst2, zeros_cpt, 2)
    h1 = _t2(m8, id2, od2, W1, b1.reshape(1, 16), 8, 16)
    m16 = _seg16(h1, widx4, dst2, zeros_cpt, 4)
    h2lo, h2hi = _t3(m16, id2, od2, W2, b2.reshape(1, 32))
    m32 = jnp.concatenate(
        [_seg16(h2lo, widx4, dst2, zeros_cpt, 4),
         _seg16(h2hi, widx4, dst2, zeros_cpt, 4)], axis=1)

    w3p = jnp.concatenate([W3[:, 0::2], W3[:, 1::2]], axis=1)
    b3p = jnp.concatenate([b3[0::2], b3[1::2]]).reshape(1, 128)
    f4wp = jnp.zeros((32, 16), _f32).at[:, :10].set(fc4W)
    f4bp = jnp.zeros((16,), _f32).at[:10].set(fc4b).reshape(1, 16)

    out = _t4(m32, id2, w3p, b3p,
              fc1W, fc1b.reshape(1, 128), fc2W, fc2b.reshape(1, 64),
              fc3W, fc3b.reshape(1, 32), f4wp, f4bp)
    return out[0, :10]
